# Initial kernel scaffold; baseline (speedup 1.0000x reference)
import jax
import jax.numpy as jnp
from jax.experimental import pallas as pl


def kernel(X_in, ones):
    raise NotImplementedError("write your pallas kernel here")


# speedup vs baseline: 1.0030x; 1.0030x over previous
import dataclasses
import functools

import jax
import jax.numpy as jnp
from jax import lax
from jax.experimental import pallas as pl
from jax.experimental.pallas import tpu as pltpu
from jax.experimental.pallas import tpu_sc as plsc

_L = 16
_NC = 2
_NS = 16
_NW = _NC * _NS
_W = 32


@functools.lru_cache(maxsize=None)
def _make_one_hot_sc(batch: int, depth: int):
    rps = batch // _NW
    nblk = rps // _W
    blk_elems = _W * depth

    mesh = plsc.VectorSubcoreMesh(core_axis_name="c", subcore_axis_name="s")

    cparams = pltpu.CompilerParams()
    if "needs_layout_passes" in pltpu.CompilerParams.__dataclass_fields__:
        cparams = dataclasses.replace(cparams, needs_layout_passes=False)

    @functools.partial(
        pl.kernel,
        out_type=jax.ShapeDtypeStruct((batch * depth,), jnp.float32),
        mesh=mesh,
        compiler_params=cparams,
        scratch_types=[
            pltpu.VMEM((rps,), jnp.int32),
            pltpu.VMEM((blk_elems,), jnp.float32),
            pltpu.VMEM((blk_elems,), jnp.float32),
            pltpu.SemaphoreType.DMA,
            pltpu.SemaphoreType.DMA,
        ],
    )
    def kern(x_hbm, o_hbm, idx_v, buf0, buf1, sem0, sem1):
        wid = lax.axis_index("s") * _NC + lax.axis_index("c")
        row0 = wid * rps
        pltpu.sync_copy(x_hbm.at[pl.ds(row0, rps)], idx_v)

        zeros = jnp.zeros((_L,), jnp.float32)
        ones = jnp.ones((_L,), jnp.float32)
        lane = lax.iota(jnp.int32, _L)
        bufs = (buf0, buf1)
        sems = (sem0, sem1)

        for buf in bufs:
            @pl.loop(0, blk_elems // _L)
            def _(i, buf=buf):
                buf[pl.ds(i * _L, _L)] = zeros

        def flat_pos(blk, half):
            cols = idx_v[pl.ds(blk * _W + half * _L, _L)]
            rows = lane + (half * _L)
            return rows * depth + cols

        for blk in range(nblk):
            buf = bufs[blk % 2]
            sem = sems[blk % 2]
            if blk >= 2:
                prev = blk - 2
                pltpu.make_async_copy(
                    buf,
                    o_hbm.at[pl.ds((row0 + prev * _W) * depth, blk_elems)],
                    sem,
                ).wait()
                for half in range(2):
                    plsc.store_scatter(buf, [flat_pos(prev, half)], zeros)
            for half in range(2):
                plsc.store_scatter(buf, [flat_pos(blk, half)], ones)
            pltpu.make_async_copy(
                buf,
                o_hbm.at[pl.ds((row0 + blk * _W) * depth, blk_elems)],
                sem,
            ).start()

        for blk in range(max(nblk - 2, 0), nblk):
            pltpu.make_async_copy(
                bufs[blk % 2],
                o_hbm.at[pl.ds((row0 + blk * _W) * depth, blk_elems)],
                sems[blk % 2],
            ).wait()

    return kern


def kernel(X_in, ones):
    batch = X_in.shape[0]
    depth = ones.shape[0]
    flat = _make_one_hot_sc(batch, depth)(X_in.astype(jnp.int32))
    return flat.reshape(batch, depth)


# speedup vs baseline: 1.7623x; 1.7570x over previous
import dataclasses
import functools

import jax
import jax.numpy as jnp
from jax import lax
from jax.experimental import pallas as pl
from jax.experimental.pallas import tpu as pltpu
from jax.experimental.pallas import tpu_sc as plsc

_L = 16
_NC = 2
_NS = 16
_NW = _NC * _NS
_W = 32


@functools.lru_cache(maxsize=None)
def _make_one_hot_sc(batch: int, depth: int):
    rps = batch // _NW
    nblk = rps // _W

    mesh = plsc.VectorSubcoreMesh(core_axis_name="c", subcore_axis_name="s")

    cparams = pltpu.CompilerParams()
    if "needs_layout_passes" in pltpu.CompilerParams.__dataclass_fields__:
        cparams = dataclasses.replace(cparams, needs_layout_passes=False)
    if "use_tc_tiling_on_sc" in pltpu.CompilerParams.__dataclass_fields__:
        cparams = dataclasses.replace(cparams, use_tc_tiling_on_sc=True)

    @functools.partial(
        pl.kernel,
        out_type=jax.ShapeDtypeStruct((batch, depth), jnp.float32),
        mesh=mesh,
        compiler_params=cparams,
        scratch_types=[
            pltpu.VMEM((rps,), jnp.int32),
            pltpu.VMEM((_W, depth), jnp.float32),
            pltpu.VMEM((_W, depth), jnp.float32),
            pltpu.SemaphoreType.DMA,
            pltpu.SemaphoreType.DMA,
        ],
    )
    def kern(x_hbm, o_hbm, idx_v, buf0, buf1, sem0, sem1):
        wid = lax.axis_index("s") * _NC + lax.axis_index("c")
        row0 = wid * rps
        pltpu.sync_copy(x_hbm.at[pl.ds(row0, rps)], idx_v)

        zeros = jnp.zeros((_L,), jnp.float32)
        ones = jnp.ones((_L,), jnp.float32)
        lane = lax.iota(jnp.int32, _L)
        bufs = (buf0, buf1)
        sems = (sem0, sem1)

        nfull = depth // _L
        for buf in bufs:
            @pl.loop(0, _W)
            def _(r, buf=buf):
                for c in range(nfull):
                    buf[r, pl.ds(c * _L, _L)] = zeros
                if nfull * _L < depth:
                    rvec = jnp.zeros((_L,), jnp.int32) + r
                    plsc.store_scatter(
                        buf, [rvec, lane + (depth - _L)], zeros)

        def scatter(buf, blk, val):
            for half in range(2):
                cols = idx_v[pl.ds(blk * _W + half * _L, _L)]
                plsc.store_scatter(buf, [lane + half * _L, cols], val)

        def dma(blk):
            return pltpu.make_async_copy(
                bufs[blk % 2],
                o_hbm.at[pl.ds(row0 + blk * _W, _W)],
                sems[blk % 2],
            )

        for blk in range(nblk):
            buf = bufs[blk % 2]
            if blk >= 2:
                dma(blk - 2).wait()
                scatter(buf, blk - 2, zeros)
            scatter(buf, blk, ones)
            dma(blk).start()

        for blk in range(max(nblk - 2, 0), nblk):
            dma(blk).wait()

    return kern


def kernel(X_in, ones):
    batch = X_in.shape[0]
    depth = ones.shape[0]
    return _make_one_hot_sc(batch, depth)(X_in.astype(jnp.int32))


# speedup vs baseline: 3.7267x; 2.1147x over previous
import dataclasses
import functools

import jax
import jax.numpy as jnp
from jax import lax
from jax.experimental import pallas as pl
from jax.experimental.pallas import tpu as pltpu
from jax.experimental.pallas import tpu_sc as plsc

_L = 16
_NC = 2
_NS = 16
_NW = _NC * _NS
_SLAB = 32
_CB = 1024


@functools.lru_cache(maxsize=None)
def _make_one_hot_sc(batch: int, depth: int):
    nblk = batch // _CB
    nvec = _CB // _L

    mesh = plsc.VectorSubcoreMesh(core_axis_name="c", subcore_axis_name="s")

    cparams = pltpu.CompilerParams()
    if "needs_layout_passes" in pltpu.CompilerParams.__dataclass_fields__:
        cparams = dataclasses.replace(cparams, needs_layout_passes=False)
    if "use_tc_tiling_on_sc" in pltpu.CompilerParams.__dataclass_fields__:
        cparams = dataclasses.replace(cparams, use_tc_tiling_on_sc=True)

    @functools.partial(
        pl.kernel,
        out_type=jax.ShapeDtypeStruct((depth, batch), jnp.float32),
        mesh=mesh,
        compiler_params=cparams,
        scratch_types=[
            pltpu.VMEM((batch,), jnp.int32),
            pltpu.VMEM((_SLAB, _CB), jnp.float32),
            pltpu.VMEM((_SLAB, _CB), jnp.float32),
            pltpu.SemaphoreType.DMA,
            pltpu.SemaphoreType.DMA,
        ],
    )
    def kern(x_hbm, o_hbm, idx_v, buf0, buf1, sem0, sem1):
        wid = lax.axis_index("s") * _NC + lax.axis_index("c")
        c0 = wid * _SLAB
        c1 = jnp.minimum(c0 + _SLAB, depth)
        pltpu.sync_copy(x_hbm, idx_v)

        zeros = jnp.zeros((_L,), jnp.float32)
        ones = jnp.ones((_L,), jnp.float32)
        izeros = jnp.zeros((_L,), jnp.int32)
        lane = lax.iota(jnp.int32, _L)
        bufs = (buf0, buf1)
        sems = (sem0, sem1)

        for buf in bufs:
            @pl.loop(0, _SLAB)
            def _(r, buf=buf):
                for c in range(_CB // _L):
                    buf[r, pl.ds(c * _L, _L)] = zeros

        def scan_scatter(buf, blk, val):
            @pl.loop(0, nvec)
            def _(j, buf=buf, blk=blk, val=val):
                v = idx_v[pl.ds(blk * _CB + j * _L, _L)]
                in_slab = (v >= (izeros + c0)) & (v < (izeros + c1))
                plsc.store_scatter(
                    buf, [v - c0, izeros + j * _L + lane], val, mask=in_slab)

        def dmas(blk):
            copies = []
            buf = bufs[blk % 2]
            for s in range(_SLAB // 8):
                copies.append((
                    c0 + 8 * s,
                    pltpu.make_async_copy(
                        buf.at[pl.ds(8 * s, 8)],
                        o_hbm.at[pl.ds(c0 + 8 * s, 8),
                                 pl.ds(blk * _CB, _CB)],
                        sems[blk % 2],
                    ),
                ))
            return copies

        for blk in range(nblk):
            buf = bufs[blk % 2]
            if blk >= 2:
                for row, cp in dmas(blk - 2):
                    @pl.when(row < depth)
                    def _(cp=cp):
                        cp.wait()
                scan_scatter(buf, blk - 2, zeros)
            scan_scatter(buf, blk, ones)
            for row, cp in dmas(blk):
                @pl.when(row < depth)
                def _(cp=cp):
                    cp.start()

        for blk in range(max(nblk - 2, 0), nblk):
            for row, cp in dmas(blk):
                @pl.when(row < depth)
                def _(cp=cp):
                    cp.wait()

    return kern


def kernel(X_in, ones):
    batch = X_in.shape[0]
    depth = ones.shape[0]
    out_t = _make_one_hot_sc(batch, depth)(X_in.astype(jnp.int32))
    return out_t.T


# speedup vs baseline: 3.8258x; 1.0266x over previous
import dataclasses
import functools

import jax
import jax.numpy as jnp
from jax import lax
from jax.experimental import pallas as pl
from jax.experimental.pallas import tpu as pltpu
from jax.experimental.pallas import tpu_sc as plsc

_L = 16
_NC = 2
_NS = 16
_NW = _NC * _NS
_SLAB = 32
_CB = 1024


@functools.lru_cache(maxsize=None)
def _make_one_hot_sc(batch: int, depth: int):
    nblk = batch // _CB
    nvec = _CB // _L

    mesh = plsc.VectorSubcoreMesh(core_axis_name="c", subcore_axis_name="s")

    cparams = pltpu.CompilerParams()
    if "needs_layout_passes" in pltpu.CompilerParams.__dataclass_fields__:
        cparams = dataclasses.replace(cparams, needs_layout_passes=False)
    if "use_tc_tiling_on_sc" in pltpu.CompilerParams.__dataclass_fields__:
        cparams = dataclasses.replace(cparams, use_tc_tiling_on_sc=True)

    @functools.partial(
        pl.kernel,
        out_type=jax.ShapeDtypeStruct((depth, batch), jnp.float32),
        mesh=mesh,
        compiler_params=cparams,
        scratch_types=[
            pltpu.VMEM((batch,), jnp.int32),
            pltpu.VMEM((_SLAB, _CB), jnp.float32),
            pltpu.VMEM((_SLAB, _CB), jnp.float32),
            pltpu.SemaphoreType.DMA,
            pltpu.SemaphoreType.DMA,
        ],
    )
    def kern(x_hbm, o_hbm, idx_v, buf0, buf1, sem0, sem1):
        wid = lax.axis_index("s") * _NC + lax.axis_index("c")
        c0 = wid * _SLAB
        c1 = jnp.minimum(c0 + _SLAB, depth)
        slab_sz = (c1 - c0).astype(jnp.uint32)
        pltpu.sync_copy(x_hbm, idx_v)

        zeros = jnp.zeros((_L,), jnp.float32)
        ones = jnp.ones((_L,), jnp.float32)
        lane = lax.iota(jnp.int32, _L)
        bufs = (buf0, buf1)
        sems = (sem0, sem1)

        for buf in bufs:
            @pl.loop(0, _SLAB)
            def _(r, buf=buf):
                for c in range(_CB // _L):
                    buf[r, pl.ds(c * _L, _L)] = zeros

        def scan_scatter(buf, blk, val):
            @pl.loop(0, nvec)
            def _(j, buf=buf, blk=blk, val=val):
                v = idx_v[pl.ds(blk * _CB + j * _L, _L)]
                rows = v - c0
                in_slab = plsc.bitcast(rows, jnp.uint32) < slab_sz
                plsc.store_scatter(
                    buf, [rows, j * _L + lane], val, mask=in_slab)

        def each_dma(buf, sem, blk, fn):
            for s in range(_SLAB // 8):
                @pl.when(c0 + 8 * s < depth)
                def _(s=s):
                    fn(pltpu.make_async_copy(
                        buf.at[pl.ds(8 * s, 8)],
                        o_hbm.at[pl.ds(c0 + 8 * s, 8),
                                 pl.ds(blk * _CB, _CB)],
                        sem,
                    ))

        def fill_and_send(buf, sem, blk):
            scan_scatter(buf, blk, ones)
            each_dma(buf, sem, blk, lambda cp: cp.start())

        def reclaim(buf, sem, blk):
            each_dma(buf, sem, blk, lambda cp: cp.wait())
            scan_scatter(buf, blk, zeros)

        for h in range(2):
            fill_and_send(bufs[h], sems[h], h)

        @pl.loop(2, nblk - 2, step=2)
        def _(g):
            for h in range(2):
                reclaim(bufs[h], sems[h], g + h - 2)
                fill_and_send(bufs[h], sems[h], g + h)

        for h in range(2):
            reclaim(bufs[h], sems[h], nblk - 4 + h)
            fill_and_send(bufs[h], sems[h], nblk - 2 + h)
        for h in range(2):
            each_dma(bufs[h], sems[h], nblk - 2 + h, lambda cp: cp.wait())

    return kern


def kernel(X_in, ones):
    batch = X_in.shape[0]
    depth = ones.shape[0]
    out_t = _make_one_hot_sc(batch, depth)(X_in.astype(jnp.int32))
    return out_t.T
